# baseline (speedup 1.0000x reference)
import functools

import jax
import jax.numpy as jnp
from jax import lax
from jax.experimental import pallas as pl
from jax.experimental.pallas import tpu as pltpu
from jax.experimental.pallas import tpu_sc as plsc

NC = 2
NS = 16
L = 16
NW = NC * NS
R = 64
N = 32768
RPW = R // NW
UN = 4
SPC = N // (L * UN)
HALF = SPC // 2


def _lane_shuffle(x, perm):
  dnums = lax.GatherDimensionNumbers(
      offset_dims=(), collapsed_slice_dims=(0,), start_index_map=(0,))
  return lax.gather(
      x, perm[:, None], dnums, slice_sizes=(1,),
      mode=lax.GatherScatterMode.PROMISE_IN_BOUNDS)


def _merge(vma, ia, vmb, ib):
  take_b = (vmb > vma) | ((vmb == vma) & (ib < ia))
  return jnp.where(take_b, vmb, vma), jnp.where(take_b, ib, ia)


def _scan_chunk(buf_v, base, lo, hi, init):
  @plsc.parallel_loop(lo, hi, unroll=4, carry=init)
  def loop(i, carry):
    vms, vss = carry
    b = base + i * (UN * L)
    ib = jnp.full((L,), i, jnp.int32)
    xs = [buf_v[pl.ds(b + k * L, L)] for k in range(UN)]
    cs = [xs[k] > vms[k] for k in range(UN)]
    new_vms = tuple(jnp.where(cs[k], xs[k], vms[k]) for k in range(UN))
    new_vss = tuple(jnp.where(cs[k], ib, vss[k]) for k in range(UN))
    return (new_vms, new_vss)

  return loop


def _finish_row(carry):
  iota = lax.broadcasted_iota(jnp.int32, (L,), 0)
  vms, vss = carry
  pairs = [(vms[k], vss[k] * (UN * L) + (k * L + iota)) for k in range(UN)]
  while len(pairs) > 1:
    nxt = []
    for j in range(0, len(pairs), 2):
      nxt.append(_merge(pairs[j][0], pairs[j][1],
                        pairs[j + 1][0], pairs[j + 1][1]))
    pairs = nxt
  vm, ii = pairs[0]
  for k in (1, 2, 4, 8):
    perm = iota ^ k
    vm2 = _lane_shuffle(vm, perm)
    ii2 = _lane_shuffle(ii, perm)
    vm, ii = _merge(vm, ii, vm2, ii2)
  return ii


@functools.partial(
    pl.kernel,
    out_type=jax.ShapeDtypeStruct((R,), jnp.int32),
    mesh=plsc.VectorSubcoreMesh(
        core_axis_name="c", subcore_axis_name="s",
        num_cores=NC, num_subcores=NS),
    scratch_types=[
        pltpu.VMEM((RPW * N,), jnp.float32),
        pltpu.VMEM((L,), jnp.int32),
        pltpu.VMEM((NS * RPW * L,), jnp.int32),
        pltpu.VMEM((NS * RPW,), jnp.int32),
        pltpu.VMEM_SHARED((NS * RPW * L,), jnp.int32),
        pltpu.SemaphoreType.DMA,
        pltpu.SemaphoreType.DMA,
        pltpu.SemaphoreType.DMA,
        pltpu.SemaphoreType.DMA,
    ],
)
def _argmax_sc(x_hbm, out_hbm, buf_v, res_v, comp_v, outb_v, stage_sh,
               sem0, sem1, sem2, sem3):
  cid = lax.axis_index("c")
  sid = lax.axis_index("s")
  wid = cid * NS + sid
  r0 = wid * RPW

  cps = []
  for r in range(RPW):
    for h in range(2):
      sem = (sem0, sem1, sem2, sem3)[r * 2 + h]
      src = x_hbm.at[r0 + r, pl.ds(h * (N // 2), N // 2)]
      dst = buf_v.at[pl.ds(r * N + h * (N // 2), N // 2)]
      cps.append(pltpu.async_copy(src, dst, sem))

  iota = lax.broadcasted_iota(jnp.int32, (L,), 0)
  neg = jnp.full((L,), -jnp.inf, jnp.float32)
  zero = jnp.zeros((L,), jnp.int32)
  init = ((neg,) * UN, (zero,) * UN)

  for r in range(RPW):
    cps[r * 2].wait()
    carry = _scan_chunk(buf_v, r * N, 0, HALF, init)
    cps[r * 2 + 1].wait()
    carry = _scan_chunk(buf_v, r * N, HALF, SPC, carry)
    res_v[...] = _finish_row(carry)
    pltpu.sync_copy(res_v, stage_sh.at[pl.ds((sid * RPW + r) * L, L)])

  plsc.subcore_barrier()

  @pl.when(sid == 0)
  def _compact():
    pltpu.sync_copy(stage_sh, comp_v)
    for half in range(RPW):
      acc = jnp.zeros((L,), jnp.int32)
      for j in range(L):
        v = comp_v[pl.ds((half * L + j) * L, L)]
        acc = jnp.where(iota == j, v, acc)
      outb_v[pl.ds(half * L, L)] = acc
    pltpu.sync_copy(outb_v, out_hbm.at[pl.ds(cid * (NS * RPW), NS * RPW)])


def kernel(inputs):
  return _argmax_sc(inputs)


# speedup vs baseline: 1.0320x; 1.0320x over previous
import functools

import jax
import jax.numpy as jnp
from jax import lax
from jax.experimental import pallas as pl
from jax.experimental.pallas import tpu as pltpu
from jax.experimental.pallas import tpu_sc as plsc

NC = 2
NS = 16
L = 16
NW = NC * NS
R = 64
N = 32768
RPW = R // NW
UN = 8
SPC = N // (L * UN)
HALF = SPC // 2


def _lane_shuffle(x, perm):
  dnums = lax.GatherDimensionNumbers(
      offset_dims=(), collapsed_slice_dims=(0,), start_index_map=(0,))
  return lax.gather(
      x, perm[:, None], dnums, slice_sizes=(1,),
      mode=lax.GatherScatterMode.PROMISE_IN_BOUNDS)


def _merge(vma, ia, vmb, ib):
  take_b = (vmb > vma) | ((vmb == vma) & (ib < ia))
  return jnp.where(take_b, vmb, vma), jnp.where(take_b, ib, ia)


def _scan_chunk(buf_v, base, lo, hi, init):
  @plsc.parallel_loop(lo, hi, unroll=4, carry=init)
  def loop(i, carry):
    vms, vss = carry
    b = base + i * (UN * L)
    ib = jnp.full((L,), i, jnp.int32)
    xs = [buf_v[pl.ds(b + k * L, L)] for k in range(UN)]
    cs = [xs[k] > vms[k] for k in range(UN)]
    new_vms = tuple(jnp.where(cs[k], xs[k], vms[k]) for k in range(UN))
    new_vss = tuple(jnp.where(cs[k], ib, vss[k]) for k in range(UN))
    return (new_vms, new_vss)

  return loop


def _finish_row(carry):
  iota = lax.broadcasted_iota(jnp.int32, (L,), 0)
  vms, vss = carry
  pairs = [(vms[k], vss[k] * (UN * L) + (k * L + iota)) for k in range(UN)]
  while len(pairs) > 1:
    nxt = []
    for j in range(0, len(pairs), 2):
      nxt.append(_merge(pairs[j][0], pairs[j][1],
                        pairs[j + 1][0], pairs[j + 1][1]))
    pairs = nxt
  vm, ii = pairs[0]
  for k in (1, 2, 4, 8):
    perm = iota ^ k
    vm2 = _lane_shuffle(vm, perm)
    ii2 = _lane_shuffle(ii, perm)
    vm, ii = _merge(vm, ii, vm2, ii2)
  return ii


@functools.partial(
    pl.kernel,
    out_type=jax.ShapeDtypeStruct((R,), jnp.int32),
    mesh=plsc.VectorSubcoreMesh(
        core_axis_name="c", subcore_axis_name="s",
        num_cores=NC, num_subcores=NS),
    scratch_types=[
        pltpu.VMEM((RPW * N,), jnp.float32),
        pltpu.VMEM((L,), jnp.int32),
        pltpu.VMEM((NS * RPW * L,), jnp.int32),
        pltpu.VMEM((NS * RPW,), jnp.int32),
        pltpu.VMEM_SHARED((NS * RPW * L,), jnp.int32),
        pltpu.SemaphoreType.DMA,
        pltpu.SemaphoreType.DMA,
        pltpu.SemaphoreType.DMA,
        pltpu.SemaphoreType.DMA,
    ],
)
def _argmax_sc(x_hbm, out_hbm, buf_v, res_v, comp_v, outb_v, stage_sh,
               sem0, sem1, sem2, sem3):
  cid = lax.axis_index("c")
  sid = lax.axis_index("s")
  wid = cid * NS + sid
  r0 = wid * RPW

  cps = []
  for r in range(RPW):
    for h in range(2):
      sem = (sem0, sem1, sem2, sem3)[r * 2 + h]
      src = x_hbm.at[r0 + r, pl.ds(h * (N // 2), N // 2)]
      dst = buf_v.at[pl.ds(r * N + h * (N // 2), N // 2)]
      cps.append(pltpu.async_copy(src, dst, sem))

  iota = lax.broadcasted_iota(jnp.int32, (L,), 0)
  neg = jnp.full((L,), -jnp.inf, jnp.float32)
  zero = jnp.zeros((L,), jnp.int32)
  init = ((neg,) * UN, (zero,) * UN)

  for r in range(RPW):
    cps[r * 2].wait()
    carry = _scan_chunk(buf_v, r * N, 0, HALF, init)
    cps[r * 2 + 1].wait()
    carry = _scan_chunk(buf_v, r * N, HALF, SPC, carry)
    res_v[...] = _finish_row(carry)
    pltpu.sync_copy(res_v, stage_sh.at[pl.ds((sid * RPW + r) * L, L)])

  plsc.subcore_barrier()

  @pl.when(sid == 0)
  def _compact():
    pltpu.sync_copy(stage_sh, comp_v)
    for half in range(RPW):
      acc = jnp.zeros((L,), jnp.int32)
      for j in range(L):
        v = comp_v[pl.ds((half * L + j) * L, L)]
        acc = jnp.where(iota == j, v, acc)
      outb_v[pl.ds(half * L, L)] = acc
    pltpu.sync_copy(outb_v, out_hbm.at[pl.ds(cid * (NS * RPW), NS * RPW)])


def kernel(inputs):
  return _argmax_sc(inputs)
